# baseline (speedup 1.0000x reference)
import functools

import jax
import jax.numpy as jnp
from jax import lax
from jax.experimental import pallas as pl
from jax.experimental.pallas import tpu as pltpu, tpu_sc as plsc

B = 2048
X_SIZE = 128
H_SIZE = 128
OUT_SIZE = 128
NUM_EMB = 16
NC, NS, L = 2, 16, 16
NW = NC * NS
TPW = B // NW
CPW = TPW // L


def _worker_id():
    return 2 * lax.axis_index("s") + lax.axis_index("c")


def _count_body(idx_hbm, cnt_hbm, idx_v, cnt_v):
    b = _worker_id()
    lane = lax.broadcasted_iota(jnp.int32, (L,), 0)
    pltpu.sync_copy(idx_hbm.at[pl.ds(b * TPW, TPW)], idx_v)
    cnt = jnp.zeros((L,), jnp.int32)
    for ci in range(CPW):
        chunk = idx_v[pl.ds(ci * L, L)]
        for e in range(NUM_EMB):
            pc = plsc.all_reduce_population_count(chunk == e)
            cnt = cnt + jnp.where(lane == e, pc, 0)
    cnt_v[...] = cnt
    pltpu.sync_copy(cnt_v, cnt_hbm.at[b])


def _dispatch_body(idx_hbm, cnt_hbm, x_hbm, dst_hbm, offs_hbm, xs_hbm,
                   idx_v, cnt_all, scan_v, dst_v, offs_v, x_v, sem):
    b = _worker_id()
    base_tok = b * TPW
    lane = lax.broadcasted_iota(jnp.int32, (L,), 0)
    x_dma = pltpu.async_copy(x_hbm.at[pl.ds(base_tok, TPW)], x_v, sem)
    pltpu.sync_copy(idx_hbm.at[pl.ds(base_tok, TPW)], idx_v)
    pltpu.sync_copy(cnt_hbm, cnt_all)
    tot = jnp.zeros((L,), jnp.int32)
    pre = jnp.zeros((L,), jnp.int32)
    for bb in range(NW):
        row = cnt_all[bb]
        tot = tot + row
        pre = pre + jnp.where(bb < b, row, 0)
    incl = tot
    for k in (1, 2, 4, 8):
        scan_v[...] = incl
        sh = plsc.load_gather(scan_v, [jnp.abs(lane - k)])
        incl = incl + jnp.where(lane >= k, sh, 0)
    excl = incl - tot
    base = excl + pre
    offs_v[...] = excl
    pltpu.sync_copy(offs_v, offs_hbm)
    x_dma.wait()
    dmas = []
    for ci in range(CPW):
        chunk = idx_v[pl.ds(ci * L, L)]
        rank = jnp.zeros((L,), jnp.int32)
        for k in range(1, L):
            shifted = plsc.load_gather(
                idx_v, [jnp.abs(lane - k) + ci * L])
            rank = rank + jnp.where((lane >= k) & (shifted == chunk), 1, 0)
        scan_v[...] = base
        selb = plsc.load_gather(scan_v, [chunk])
        dst_c = selb + rank
        dst_v[pl.ds(ci * L, L)] = dst_c
        dmas.append(pltpu.async_copy(
            x_v.at[pl.ds(ci * L, L)], xs_hbm.at[dst_c], sem))
        for e in range(NUM_EMB):
            pc = plsc.all_reduce_population_count(chunk == e)
            base = base + jnp.where(lane == e, pc, 0)
    pltpu.sync_copy(dst_v, dst_hbm.at[pl.ds(base_tok, TPW)])
    for d in dmas:
        d.wait()


def _unsort_body(dst_hbm, ys_hbm, out_hbm, dst_v, y_v, sem):
    b = _worker_id()
    base_tok = b * TPW
    pltpu.sync_copy(dst_hbm.at[pl.ds(base_tok, TPW)], dst_v)
    pltpu.async_copy(ys_hbm.at[dst_v], y_v, sem).wait()
    pltpu.sync_copy(y_v, out_hbm.at[pl.ds(base_tok, TPW)])


@functools.cache
def _sc_kernels():
    mesh = plsc.VectorSubcoreMesh(core_axis_name="c", subcore_axis_name="s",
                                  num_cores=NC, num_subcores=NS)
    params = pltpu.CompilerParams(needs_layout_passes=False)
    count = pl.kernel(
        _count_body,
        out_type=jax.ShapeDtypeStruct((NW, NUM_EMB), jnp.int32),
        mesh=mesh,
        scratch_types=[pltpu.VMEM((TPW,), jnp.int32),
                       pltpu.VMEM((NUM_EMB,), jnp.int32)],
        compiler_params=params,
    )
    dispatch = pl.kernel(
        _dispatch_body,
        out_type=[
            jax.ShapeDtypeStruct((B,), jnp.int32),
            jax.ShapeDtypeStruct((NUM_EMB,), jnp.int32),
            jax.ShapeDtypeStruct((B, X_SIZE), jnp.float32),
        ],
        mesh=mesh,
        scratch_types=[
            pltpu.VMEM((TPW,), jnp.int32),
            pltpu.VMEM((NW, NUM_EMB), jnp.int32),
            pltpu.VMEM((NUM_EMB,), jnp.int32),
            pltpu.VMEM((TPW,), jnp.int32),
            pltpu.VMEM((NUM_EMB,), jnp.int32),
            pltpu.VMEM((TPW, X_SIZE), jnp.float32),
            pltpu.SemaphoreType.DMA,
        ],
        compiler_params=params,
    )
    unsort = pl.kernel(
        _unsort_body,
        out_type=jax.ShapeDtypeStruct((B, OUT_SIZE), jnp.float32),
        mesh=mesh,
        scratch_types=[
            pltpu.VMEM((TPW,), jnp.int32),
            pltpu.VMEM((TPW, OUT_SIZE), jnp.float32),
            pltpu.SemaphoreType.DMA,
        ],
        compiler_params=params,
    )
    return count, dispatch, unsort


def _tc_body(offs_ref, xs_ref, w1_ref, b1_ref, w2_ref, b2_ref, ys_ref):
    t = pl.program_id(0)
    nrows = B // NUM_EMB
    row0 = t * nrows
    e_lo = jnp.int32(0)
    e_hi = jnp.int32(0)
    for i in range(NUM_EMB):
        o = offs_ref[i]
        e_lo += jnp.where(o <= row0, 1, 0)
        e_hi += jnp.where(o < row0 + nrows, 1, 0)
    e_lo -= 1
    e_hi -= 1
    xb = xs_ref[...].astype(jnp.bfloat16)
    ys_ref[...] = jnp.zeros_like(ys_ref)
    rowid = lax.broadcasted_iota(jnp.int32, (nrows, 1), 0)

    def span(e, _):
        s = offs_ref[e]
        nxt = offs_ref[jnp.minimum(e + 1, NUM_EMB - 1)]
        end = jnp.where(e == NUM_EMB - 1, B, nxt)
        rmask = (rowid >= s - row0) & (rowid < end - row0)
        w1e = w1_ref[e]
        h = lax.dot_general(xb, w1e, (((1,), (1,)), ((), ())),
                            preferred_element_type=jnp.float32)
        h = jnp.maximum(h + b1_ref[e], 0.0).astype(jnp.bfloat16)
        w2e = w2_ref[e]
        y = lax.dot_general(h, w2e, (((1,), (1,)), ((), ())),
                            preferred_element_type=jnp.float32)
        y = y + b2_ref[e]
        ys_ref[...] += jnp.where(rmask, y, 0.0)
        return 0

    lax.fori_loop(e_lo, e_hi + 1, span, 0)


def _tc_mlp(offs, xs, W1b, b1r, W2b, b2r):
    nrows = B // NUM_EMB
    return pl.pallas_call(
        _tc_body,
        grid=(NUM_EMB,),
        in_specs=[
            pl.BlockSpec(memory_space=pltpu.SMEM),
            pl.BlockSpec((nrows, X_SIZE), lambda t: (t, 0)),
            pl.BlockSpec((NUM_EMB, H_SIZE, X_SIZE), lambda t: (0, 0, 0)),
            pl.BlockSpec((NUM_EMB, 1, H_SIZE), lambda t: (0, 0, 0)),
            pl.BlockSpec((NUM_EMB, OUT_SIZE, H_SIZE), lambda t: (0, 0, 0)),
            pl.BlockSpec((NUM_EMB, 1, OUT_SIZE), lambda t: (0, 0, 0)),
        ],
        out_specs=pl.BlockSpec((nrows, OUT_SIZE), lambda t: (t, 0)),
        out_shape=jax.ShapeDtypeStruct((B, OUT_SIZE), jnp.float32),
    )(offs, xs, W1b, b1r, W2b, b2r)


@jax.jit
def _run(emb_idx, x, W1, b1, W2, b2):
    count, dispatch, unsort = _sc_kernels()
    cnt = count(emb_idx)
    dst, offs, xs = dispatch(emb_idx, cnt, x)
    ys = _tc_mlp(offs, xs,
                 W1.astype(jnp.bfloat16), b1.reshape(NUM_EMB, 1, H_SIZE),
                 W2.astype(jnp.bfloat16), b2.reshape(NUM_EMB, 1, OUT_SIZE))
    return unsort(dst, ys)


def kernel(quant_fn, x, emb_idx, W1, b1, W2, b2):
    del quant_fn
    return _run(emb_idx, x, W1, b1, W2, b2)
